# Initial kernel scaffold; baseline (speedup 1.0000x reference)
import jax
import jax.numpy as jnp
from jax.experimental import pallas as pl


def kernel(x, h, y, p_xy, p_xh, p_hy):
    raise NotImplementedError("write your pallas kernel here")


# speedup vs baseline: 27.5539x; 27.5539x over previous
import functools

import jax
import jax.numpy as jnp
from jax import lax
from jax.experimental import pallas as pl
from jax.experimental.pallas import tpu as pltpu
from jax.experimental.pallas import tpu_sc as plsc

N = 4096
NSQ = N * N
K_ACTIVE = 838861
LR = 0.01

CB = 256


def _stage_a_body(pxy_ref, pxh_ref, phy_ref, xf_ref, hf_ref, hp_ref, yp_ref,
                  out_ref):
    dn = (((0,), (0,)), ((), ()))
    pres = (xf_ref[...], xf_ref[...], hf_ref[...])
    posts = (yp_ref[...], hp_ref[...], yp_ref[...])
    ps = (pxy_ref, pxh_ref, phy_ref)
    for m in range(3):
        u = lax.dot_general(pres[m], posts[m], dimension_numbers=dn,
                            preferred_element_type=jnp.float32)
        w = ps[m][...] + LR * u
        c = jnp.sum(w, axis=0, keepdims=True)
        out_ref[m] = w / c


def _stage_a(p_xy, p_xh, p_hy, x, h, y):
    xb = x.astype(jnp.bfloat16)
    hb = h.astype(jnp.bfloat16)
    yb = y.astype(jnp.bfloat16)
    grid = (N // CB,)
    p_spec = pl.BlockSpec((N, CB), lambda j: (0, j))
    full_spec = pl.BlockSpec((64, N), lambda j: (0, 0))
    post_spec = pl.BlockSpec((64, CB), lambda j: (0, j))
    return pl.pallas_call(
        _stage_a_body,
        grid=grid,
        in_specs=[p_spec, p_spec, p_spec, full_spec, full_spec, post_spec,
                  post_spec],
        out_specs=pl.BlockSpec((3, N, CB), lambda j: (0, 0, j)),
        out_shape=jax.ShapeDtypeStruct((3, N, N), jnp.float32),
    )(p_xy, p_xh, p_hy, xb, hb, hb, yb)


NWORKERS = 32
SHARD = NSQ // NWORKERS
CHUNK = 16384
NCHUNK = SHARD // CHUNK


def _make_sc_hist(shift, nbins, mshift):
    mesh = plsc.VectorSubcoreMesh(core_axis_name="c", subcore_axis_name="s")

    @functools.partial(
        pl.kernel,
        mesh=mesh,
        compiler_params=pltpu.CompilerParams(needs_layout_passes=False),
        out_type=jax.ShapeDtypeStruct((3 * NWORKERS * nbins * 16,), jnp.int32),
        scratch_types=[
            pltpu.VMEM((CHUNK,), jnp.float32),
            pltpu.VMEM((16,), jnp.int32),
            pltpu.VMEM((nbins * 16,), jnp.int32),
        ],
    )
    def hist_kernel(v_hbm, mval_hbm, out_hbm, buf, mvbuf, hist):
        wid = lax.axis_index("s") * 2 + lax.axis_index("c")
        base = wid * SHARD
        lanes = lax.iota(jnp.int32, 16)
        ones = jnp.full((16,), 1, jnp.int32)
        zeros16 = jnp.zeros((16,), jnp.int32)
        for m in range(3):
            def zero_body(b, carry):
                hist[pl.ds(b * 16, 16)] = zeros16
                return carry
            lax.fori_loop(0, nbins, zero_body, 0)
            pltpu.sync_copy(mval_hbm.at[pl.ds(m * 16, 16)], mvbuf)
            mv = mvbuf[...]

            def chunk_body(ci, carry):
                pltpu.sync_copy(
                    v_hbm.at[pl.ds(m * NSQ + base + ci * CHUNK, CHUNK)], buf)

                def vec_body(i, c2):
                    vals = buf[pl.ds(i * 16, 16)]
                    bits = lax.bitcast_convert_type(vals, jnp.int32)
                    bin_ = jnp.bitwise_and(
                        lax.shift_right_logical(bits, shift), nbins - 1)
                    idx = bin_ * 16 + lanes
                    msk = lax.shift_right_logical(bits, mshift) == mv
                    plsc.addupdate_scatter(hist, [idx], ones, mask=msk)
                    return c2

                lax.fori_loop(0, CHUNK // 16, vec_body, 0)
                return carry

            lax.fori_loop(0, NCHUNK, chunk_body, 0)
            pltpu.sync_copy(
                hist,
                out_hbm.at[pl.ds((m * NWORKERS + wid) * nbins * 16,
                                 nbins * 16)])

    return hist_kernel


_sc_hist_l1 = _make_sc_hist(shift=20, nbins=2048, mshift=31)
_sc_hist_l2 = _make_sc_hist(shift=8, nbins=4096, mshift=20)
_sc_hist_l3 = _make_sc_hist(shift=0, nbins=256, mshift=8)


def _pick(hist_dump, nbins, krem):
    h = hist_dump.reshape(3, NWORKERS, nbins, 16).sum(axis=(1, 3))
    above = (jnp.cumsum(h[:, ::-1], axis=1)[:, ::-1] - h)
    sel = (above < krem[:, None]) & (above + h >= krem[:, None])
    b = jnp.argmax(sel, axis=1).astype(jnp.int32)
    krem2 = krem - jnp.take_along_axis(above, b[:, None].astype(jnp.int32),
                                       axis=1)[:, 0]
    return b, krem2


def _stage_h_body(t_ref, v_ref, out_ref):
    for m in range(3):
        vm = v_ref[m]
        out_ref[m] = jnp.where(vm >= t_ref[m], vm, 0.0)


def _stage_h(v, t):
    grid = (N // CB,)
    return pl.pallas_call(
        _stage_h_body,
        grid=grid,
        in_specs=[
            pl.BlockSpec(memory_space=pltpu.SMEM),
            pl.BlockSpec((3, N, CB), lambda j: (0, 0, j)),
        ],
        out_specs=pl.BlockSpec((3, N, CB), lambda j: (0, 0, j)),
        out_shape=jax.ShapeDtypeStruct((3, N, N), jnp.float32),
    )(t, v)


def kernel(x, h, y, p_xy, p_xh, p_hy):
    v = _stage_a(p_xy, p_xh, p_hy, x, h, y)
    v_flat = v.reshape(3 * NSQ)

    k0 = jnp.full((3,), K_ACTIVE, jnp.int32)
    mval1 = jnp.zeros((48,), jnp.int32)
    d1 = _sc_hist_l1(v_flat, mval1)
    b1, k1 = _pick(d1, 2048, k0)

    mval2 = jnp.broadcast_to(b1[:, None], (3, 16)).reshape(48).astype(jnp.int32)
    d2 = _sc_hist_l2(v_flat, mval2)
    b2, k2 = _pick(d2, 4096, k1)

    b12 = (b1 << 12) | b2
    mval3 = jnp.broadcast_to(b12[:, None], (3, 16)).reshape(48).astype(jnp.int32)
    d3 = _sc_hist_l3(v_flat, mval3)
    b3, _ = _pick(d3, 256, k2)

    t_bits = (b1 << 20) | (b2 << 8) | b3
    t = lax.bitcast_convert_type(t_bits, jnp.float32)
    return _stage_h(v, t)


# speedup vs baseline: 35.3254x; 1.2821x over previous
import functools

import jax
import jax.numpy as jnp
from jax import lax
from jax.experimental import pallas as pl
from jax.experimental.pallas import tpu as pltpu
from jax.experimental.pallas import tpu_sc as plsc

N = 4096
NSQ = N * N
K_ACTIVE = 838861
LR = 0.01

CB = 256


def _stage_a_body(pxy_ref, pxh_ref, phy_ref, xf_ref, hf_ref, hp_ref, yp_ref,
                  out_ref):
    dn = (((0,), (0,)), ((), ()))
    pres = (xf_ref[...], xf_ref[...], hf_ref[...])
    posts = (yp_ref[...], hp_ref[...], yp_ref[...])
    ps = (pxy_ref, pxh_ref, phy_ref)
    for m in range(3):
        u = lax.dot_general(pres[m], posts[m], dimension_numbers=dn,
                            preferred_element_type=jnp.float32)
        w = ps[m][...] + LR * u
        c = jnp.sum(w, axis=0, keepdims=True)
        out_ref[m] = w / c


def _stage_a(p_xy, p_xh, p_hy, x, h, y):
    xb = x.astype(jnp.bfloat16)
    hb = h.astype(jnp.bfloat16)
    yb = y.astype(jnp.bfloat16)
    grid = (N // CB,)
    p_spec = pl.BlockSpec((N, CB), lambda j: (0, j))
    full_spec = pl.BlockSpec((64, N), lambda j: (0, 0))
    post_spec = pl.BlockSpec((64, CB), lambda j: (0, j))
    return pl.pallas_call(
        _stage_a_body,
        grid=grid,
        in_specs=[p_spec, p_spec, p_spec, full_spec, full_spec, post_spec,
                  post_spec],
        out_specs=pl.BlockSpec((3, N, CB), lambda j: (0, 0, j)),
        out_shape=jax.ShapeDtypeStruct((3, N, N), jnp.float32),
    )(p_xy, p_xh, p_hy, xb, hb, hb, yb)


NWORKERS = 32
SHARD = NSQ // NWORKERS
CHUNK = 16384
NCHUNK = SHARD // CHUNK


UNROLL = 8


def _make_sc_hist(shift, nbins, mshift):
    mesh = plsc.VectorSubcoreMesh(core_axis_name="c", subcore_axis_name="s")

    @functools.partial(
        pl.kernel,
        mesh=mesh,
        compiler_params=pltpu.CompilerParams(needs_layout_passes=False),
        out_type=jax.ShapeDtypeStruct((3 * NWORKERS * nbins * 16,), jnp.int32),
        scratch_types=[
            pltpu.VMEM((CHUNK,), jnp.float32),
            pltpu.VMEM((CHUNK,), jnp.float32),
            pltpu.VMEM((16,), jnp.int32),
            pltpu.VMEM((nbins * 16,), jnp.int32),
            pltpu.SemaphoreType.DMA,
            pltpu.SemaphoreType.DMA,
        ],
    )
    def hist_kernel(v_hbm, mval_hbm, out_hbm, bufa, bufb, mvbuf, hist,
                    sema, semb):
        wid = lax.axis_index("s") * 2 + lax.axis_index("c")
        base = wid * SHARD
        lanes = lax.iota(jnp.int32, 16)
        lanebase = lanes * nbins
        ones = jnp.full((16,), 1, jnp.int32)
        zeros16 = jnp.zeros((16,), jnp.int32)

        def process(buf, mv):
            def vec_body(i, c2):
                for u in range(UNROLL):
                    vals = buf[pl.ds((i * UNROLL + u) * 16, 16)]
                    bits = lax.bitcast_convert_type(vals, jnp.int32)
                    if shift == 0:
                        bin_ = jnp.bitwise_and(bits, nbins - 1)
                    elif mshift is None:
                        bin_ = lax.shift_right_logical(bits, shift)
                    else:
                        bin_ = jnp.bitwise_and(
                            lax.shift_right_logical(bits, shift), nbins - 1)
                    idx = lanebase + bin_
                    if mshift is None:
                        plsc.addupdate_scatter(hist, [idx], ones)
                    else:
                        msk = lax.shift_right_logical(bits, mshift) == mv
                        plsc.addupdate_scatter(hist, [idx], ones, mask=msk)
                return c2

            lax.fori_loop(0, (CHUNK // 16) // UNROLL, vec_body, 0)

        for m in range(3):
            def zero_body(b, carry):
                for u in range(UNROLL):
                    hist[pl.ds((b * UNROLL + u) * 16, 16)] = zeros16
                return carry
            lax.fori_loop(0, nbins // UNROLL, zero_body, 0)
            pltpu.sync_copy(mval_hbm.at[pl.ds(m * 16, 16)], mvbuf)
            mv = mvbuf[...]
            moff = m * NSQ + base
            pltpu.async_copy(v_hbm.at[pl.ds(moff, CHUNK)], bufa, sema)

            def pair_body(j, carry):
                pltpu.async_copy(
                    v_hbm.at[pl.ds(moff + (2 * j + 1) * CHUNK, CHUNK)],
                    bufb, semb)
                pltpu.make_async_copy(
                    v_hbm.at[pl.ds(moff, CHUNK)], bufa, sema).wait()
                process(bufa, mv)

                @pl.when(2 * j + 2 < NCHUNK)
                def _prefetch():
                    pltpu.async_copy(
                        v_hbm.at[pl.ds(moff + (2 * j + 2) * CHUNK, CHUNK)],
                        bufa, sema)

                pltpu.make_async_copy(
                    v_hbm.at[pl.ds(moff, CHUNK)], bufb, semb).wait()
                process(bufb, mv)
                return carry

            lax.fori_loop(0, NCHUNK // 2, pair_body, 0)
            pltpu.sync_copy(
                hist,
                out_hbm.at[pl.ds((m * NWORKERS + wid) * nbins * 16,
                                 nbins * 16)])

    return hist_kernel


_sc_hist_l1 = _make_sc_hist(shift=20, nbins=2048, mshift=None)
_sc_hist_l2 = _make_sc_hist(shift=8, nbins=4096, mshift=20)
_sc_hist_l3 = _make_sc_hist(shift=0, nbins=256, mshift=8)


def _pick(hist_dump, nbins, krem):
    h = hist_dump.reshape(3, NWORKERS, 16, nbins).sum(axis=(1, 2))
    above = (jnp.cumsum(h[:, ::-1], axis=1)[:, ::-1] - h)
    sel = (above < krem[:, None]) & (above + h >= krem[:, None])
    b = jnp.argmax(sel, axis=1).astype(jnp.int32)
    krem2 = krem - jnp.take_along_axis(above, b[:, None].astype(jnp.int32),
                                       axis=1)[:, 0]
    return b, krem2


def _stage_h_body(t_ref, v_ref, out_ref):
    for m in range(3):
        vm = v_ref[m]
        out_ref[m] = jnp.where(vm >= t_ref[m], vm, 0.0)


def _stage_h(v, t):
    grid = (N // CB,)
    return pl.pallas_call(
        _stage_h_body,
        grid=grid,
        in_specs=[
            pl.BlockSpec(memory_space=pltpu.SMEM),
            pl.BlockSpec((3, N, CB), lambda j: (0, 0, j)),
        ],
        out_specs=pl.BlockSpec((3, N, CB), lambda j: (0, 0, j)),
        out_shape=jax.ShapeDtypeStruct((3, N, N), jnp.float32),
    )(t, v)


def kernel(x, h, y, p_xy, p_xh, p_hy):
    v = _stage_a(p_xy, p_xh, p_hy, x, h, y)
    v_flat = v.reshape(3 * NSQ)

    k0 = jnp.full((3,), K_ACTIVE, jnp.int32)
    mval1 = jnp.zeros((48,), jnp.int32)
    d1 = _sc_hist_l1(v_flat, mval1)
    b1, k1 = _pick(d1, 2048, k0)

    mval2 = jnp.broadcast_to(b1[:, None], (3, 16)).reshape(48).astype(jnp.int32)
    d2 = _sc_hist_l2(v_flat, mval2)
    b2, k2 = _pick(d2, 4096, k1)

    b12 = (b1 << 12) | b2
    mval3 = jnp.broadcast_to(b12[:, None], (3, 16)).reshape(48).astype(jnp.int32)
    d3 = _sc_hist_l3(v_flat, mval3)
    b3, _ = _pick(d3, 256, k2)

    t_bits = (b1 << 20) | (b2 << 8) | b3
    t = lax.bitcast_convert_type(t_bits, jnp.float32)
    return _stage_h(v, t)
